# Initial kernel scaffold; baseline (speedup 1.0000x reference)
import jax
import jax.numpy as jnp
from jax.experimental import pallas as pl


def kernel(edge_index, user_emb, item_emb):
    raise NotImplementedError("write your pallas kernel here")


# speedup vs baseline: 19.0342x; 19.0342x over previous
import jax
import jax.numpy as jnp
from jax import lax
from jax.experimental import pallas as pl
from jax.experimental.pallas import tpu as pltpu
from jax.experimental.pallas import tpu_sc as plsc

N_USER = 50000
N_ITEM = 50000
EMB = 32
N_EDGES = 1600000

NT = 16
LANES = 16
N_PAD = 51200
NODES_PER_TILE = N_PAD // NT
WB_CHUNK = 64
CHUNK = 128
GROUP = 4
DEG_GROUP = 8
N_CHUNKS = 12544
E_PAD = N_CHUNKS * CHUNK
CHUNKS_PER_TILE = N_CHUNKS // NT
GROUPS_PER_TILE = CHUNKS_PER_TILE // GROUP
DEG_GROUPS_PER_TILE = CHUNKS_PER_TILE // DEG_GROUP
PAD_NODE = N_PAD - 1

_mesh = plsc.VectorSubcoreMesh(core_axis_name="c", subcore_axis_name="s")

_f32 = jnp.float32
_zeros16 = lambda: jnp.zeros((LANES,), _f32)


def _deg_body(edges2, recs_out, idx_buf, ones_buf, red_buf, out_buf,
              deg_acc, sem_s):
    c = lax.axis_index("c")
    s = lax.axis_index("s")
    nb = s * NODES_PER_TILE
    cnt = 1 - c

    def fill_ones(j, carry):
        ones_buf[pl.ds(j * LANES, LANES)] = jnp.ones((LANES,), _f32)
        return carry
    lax.fori_loop(0, CHUNK // LANES, fill_ones, 0)

    def fill_zero(j, carry):
        out_buf[pl.ds(j * LANES, LANES)] = _zeros16()
        return carry
    lax.fori_loop(0, NODES_PER_TILE // LANES, fill_zero, 0)
    pltpu.sync_copy(out_buf, deg_acc.at[pl.ds(nb, NODES_PER_TILE)])
    plsc.subcore_barrier()

    def group_body(g, carry):
        base = s * CHUNKS_PER_TILE + g * DEG_GROUP
        pltpu.sync_copy(edges2.at[cnt, pl.ds(base, DEG_GROUP)], idx_buf)
        cps = [pltpu.async_copy(ones_buf, deg_acc.at[idx_buf.at[j]],
                                sem_s, add=True)
               for j in range(DEG_GROUP)]
        for cp in cps:
            cp.wait()
        return carry
    lax.fori_loop(0, DEG_GROUPS_PER_TILE, group_body, 0)
    plsc.subcore_barrier()

    pltpu.sync_copy(deg_acc.at[pl.ds(nb, NODES_PER_TILE)], red_buf)

    def red_body(j, carry):
        tot = red_buf[pl.ds(j * LANES, LANES)]
        out_buf[pl.ds(j * LANES, LANES)] = 1.0 / jnp.maximum(tot, 1.0)
        return carry
    lax.fori_loop(0, NODES_PER_TILE // LANES, red_body, 0)
    pltpu.sync_copy(out_buf, recs_out.at[pl.ds(cnt * N_PAD + nb, NODES_PER_TILE)])


def _make_prop_body(final_mode):
    def body(*refs):
        if final_mode:
            (edges2, tabs, recs, bases, outs,
             acc, gidx, sidx, rows, wb, rec_tile, h0b, h1b,
             sem_g, sem_s) = refs
        else:
            (edges2, tabs, recs, outs,
             acc, gidx, sidx, rows, wb, rec_tile,
             sem_g, sem_s) = refs
            bases = h0b = h1b = None

        c = lax.axis_index("c")
        s = lax.axis_index("s")
        gd = c
        sd = 1 - c

        def zb(n, carry):
            wb[n, pl.ds(0, LANES)] = _zeros16()
            wb[n, pl.ds(LANES, LANES)] = _zeros16()
            return carry
        lax.fori_loop(0, WB_CHUNK, zb, 0)

        def zacc(q, carry):
            pltpu.sync_copy(
                wb, acc.at[pl.ds(s * NODES_PER_TILE + q * WB_CHUNK, WB_CHUNK)])
            return carry
        lax.fori_loop(0, NODES_PER_TILE // WB_CHUNK, zacc, 0)
        plsc.subcore_barrier()

        def group_body(g, carry):
            base = s * CHUNKS_PER_TILE + g * GROUP
            pltpu.sync_copy(edges2.at[gd, pl.ds(base, GROUP)], gidx)
            pltpu.sync_copy(edges2.at[sd, pl.ds(base, GROUP)], sidx)
            cps = [pltpu.async_copy(tabs.at[gd].at[gidx.at[j]], rows.at[j],
                                    sem_g)
                   for j in range(GROUP)]
            for cp in cps:
                cp.wait()
            cps2 = [pltpu.async_copy(rows.at[j], acc.at[sidx.at[j]], sem_s,
                                     add=True)
                    for j in range(GROUP)]
            for cp in cps2:
                cp.wait()
            return carry
        lax.fori_loop(0, GROUPS_PER_TILE, group_body, 0)
        plsc.subcore_barrier()

        pltpu.sync_copy(
            recs.at[pl.ds(sd * N_PAD + s * NODES_PER_TILE, NODES_PER_TILE)],
            rec_tile)

        def wq(q, carry):
            nb = s * NODES_PER_TILE + q * WB_CHUNK
            pltpu.sync_copy(acc.at[pl.ds(nb, WB_CHUNK)], wb)
            if final_mode:
                pltpu.sync_copy(bases.at[sd, pl.ds(nb, WB_CHUNK)], h0b)
                pltpu.sync_copy(tabs.at[sd, pl.ds(nb, WB_CHUNK)], h1b)

            def wn(j, carry2):
                rv = rec_tile[pl.ds(q * WB_CHUNK + j * LANES, LANES)]
                for k in range(LANES):
                    n = j * LANES + k
                    r = rv[k]
                    v0 = wb[n, pl.ds(0, LANES)] * r
                    v1 = wb[n, pl.ds(LANES, LANES)] * r
                    if final_mode:
                        third = _f32(1.0 / 3.0)
                        v0 = (v0 + h0b[n, pl.ds(0, LANES)]
                              + h1b[n, pl.ds(0, LANES)]) * third
                        v1 = (v1 + h0b[n, pl.ds(LANES, LANES)]
                              + h1b[n, pl.ds(LANES, LANES)]) * third
                    wb[n, pl.ds(0, LANES)] = v0
                    wb[n, pl.ds(LANES, LANES)] = v1
                return carry2
            lax.fori_loop(0, WB_CHUNK // LANES, wn, 0)
            pltpu.sync_copy(wb, outs.at[sd, pl.ds(nb, WB_CHUNK)])
            return carry
        lax.fori_loop(0, NODES_PER_TILE // WB_CHUNK, wq, 0)

    return body


_deg_kernel = pl.kernel(
    _deg_body,
    out_type=jax.ShapeDtypeStruct((2 * N_PAD,), _f32),
    mesh=_mesh,
    compiler_params=pltpu.CompilerParams(use_tc_tiling_on_sc=False),
    scratch_types=[
        pltpu.VMEM((DEG_GROUP, CHUNK), jnp.int32),
        pltpu.VMEM((CHUNK,), jnp.float32),
        pltpu.VMEM((NODES_PER_TILE,), jnp.float32),
        pltpu.VMEM((NODES_PER_TILE,), jnp.float32),
        pltpu.VMEM_SHARED((N_PAD,), jnp.float32),
        pltpu.SemaphoreType.DMA,
    ],
)

_prop_scratch = [
    pltpu.VMEM_SHARED((N_PAD, EMB), jnp.float32),
    pltpu.VMEM((GROUP, CHUNK), jnp.int32),
    pltpu.VMEM((GROUP, CHUNK), jnp.int32),
    pltpu.VMEM((GROUP, CHUNK, EMB), jnp.float32),
    pltpu.VMEM((WB_CHUNK, EMB), jnp.float32),
    pltpu.VMEM((NODES_PER_TILE,), jnp.float32),
]

_prop_kernel = pl.kernel(
    _make_prop_body(False),
    out_type=jax.ShapeDtypeStruct((2, N_PAD, EMB), _f32),
    mesh=_mesh,
    compiler_params=pltpu.CompilerParams(use_tc_tiling_on_sc=False),
    scratch_types=_prop_scratch + [
        pltpu.SemaphoreType.DMA,
        pltpu.SemaphoreType.DMA,
    ],
)

_final_kernel = pl.kernel(
    _make_prop_body(True),
    out_type=jax.ShapeDtypeStruct((2, N_PAD, EMB), _f32),
    mesh=_mesh,
    compiler_params=pltpu.CompilerParams(use_tc_tiling_on_sc=False),
    scratch_types=_prop_scratch + [
        pltpu.VMEM((WB_CHUNK, EMB), jnp.float32),
        pltpu.VMEM((WB_CHUNK, EMB), jnp.float32),
        pltpu.SemaphoreType.DMA,
        pltpu.SemaphoreType.DMA,
    ],
)


def kernel(edge_index, user_emb, item_emb):
    pad = jnp.full((2, E_PAD - N_EDGES), PAD_NODE, jnp.int32)
    edges2 = jnp.concatenate([edge_index, pad], axis=1)
    edges2 = edges2.reshape(2, N_CHUNKS, CHUNK)
    zrow = jnp.zeros((1, N_PAD - N_USER, EMB), _f32)
    tabs0 = jnp.concatenate(
        [jnp.stack([user_emb, item_emb]),
         jnp.concatenate([zrow, zrow])], axis=1)

    recs = _deg_kernel(edges2)
    tabs1 = _prop_kernel(edges2, tabs0, recs)
    outs = _final_kernel(edges2, tabs1, recs, tabs0)
    return outs[0, :N_USER], outs[1, :N_ITEM]


# speedup vs baseline: 27.2930x; 1.4339x over previous
import jax
import jax.numpy as jnp
from jax import lax
from jax.experimental import pallas as pl
from jax.experimental.pallas import tpu as pltpu
from jax.experimental.pallas import tpu_sc as plsc

N_USER = 50000
N_ITEM = 50000
EMB = 32
N_EDGES = 1600000

NT = 16
LANES = 16
N_PAD = 51200
NODES_PER_TILE = N_PAD // NT
WB_CHUNK = 32
CHUNK = 128
NSLOT = 4
LOOKAHEAD = 2
SG = 8
DEG_GROUP = 8
N_CHUNKS = 12544
E_PAD = N_CHUNKS * CHUNK
CHUNKS_PER_TILE = N_CHUNKS // NT
SUPERS_PER_TILE = CHUNKS_PER_TILE // SG
ITERS_PER_TILE = CHUNKS_PER_TILE // NSLOT
DEG_GROUPS_PER_TILE = CHUNKS_PER_TILE // DEG_GROUP
PAD_NODE = N_PAD - 1

_mesh = plsc.VectorSubcoreMesh(core_axis_name="c", subcore_axis_name="s")

_f32 = jnp.float32
_zeros16 = lambda: jnp.zeros((LANES,), _f32)


def _deg_body(edges2, recs_out, idx_buf, ones_buf, red_buf, out_buf,
              deg_acc, sem_s):
    c = lax.axis_index("c")
    s = lax.axis_index("s")
    nb = s * NODES_PER_TILE
    cnt = 1 - c

    def fill_ones(j, carry):
        ones_buf[pl.ds(j * LANES, LANES)] = jnp.ones((LANES,), _f32)
        return carry
    lax.fori_loop(0, CHUNK // LANES, fill_ones, 0)

    def fill_zero(j, carry):
        out_buf[pl.ds(j * LANES, LANES)] = _zeros16()
        return carry
    lax.fori_loop(0, NODES_PER_TILE // LANES, fill_zero, 0)
    pltpu.sync_copy(out_buf, deg_acc.at[pl.ds(nb, NODES_PER_TILE)])
    plsc.subcore_barrier()

    def group_body(g, carry):
        base = s * CHUNKS_PER_TILE + g * DEG_GROUP
        pltpu.sync_copy(edges2.at[cnt, pl.ds(base, DEG_GROUP)], idx_buf)
        cps = [pltpu.async_copy(ones_buf, deg_acc.at[idx_buf.at[j]],
                                sem_s, add=True)
               for j in range(DEG_GROUP)]
        for cp in cps:
            cp.wait()
        return carry
    lax.fori_loop(0, DEG_GROUPS_PER_TILE, group_body, 0)
    plsc.subcore_barrier()

    pltpu.sync_copy(deg_acc.at[pl.ds(nb, NODES_PER_TILE)], red_buf)

    def red_body(j, carry):
        tot = red_buf[pl.ds(j * LANES, LANES)]
        out_buf[pl.ds(j * LANES, LANES)] = 1.0 / jnp.maximum(tot, 1.0)
        return carry
    lax.fori_loop(0, NODES_PER_TILE // LANES, red_body, 0)
    pltpu.sync_copy(out_buf, recs_out.at[pl.ds(cnt * N_PAD + nb, NODES_PER_TILE)])


def _make_prop_body(final_mode):
    def body(*refs):
        if final_mode:
            (edges2, tabs, recs, bases, outs,
             acc, gidx3, sidx3, rows, wb, rec_buf, h0b, h1b,
             sem_i, *sems) = refs
        else:
            (edges2, tabs, recs, outs,
             acc, gidx3, sidx3, rows, wb, rec_buf,
             sem_i, *sems) = refs
            bases = h0b = h1b = None
        sems_g = sems[:NSLOT]
        sems_s = sems[NSLOT:]

        c = lax.axis_index("c")
        s = lax.axis_index("s")
        gd = c
        sd = 1 - c

        def zb(n, carry):
            wb[n, pl.ds(0, LANES)] = _zeros16()
            wb[n, pl.ds(LANES, LANES)] = _zeros16()
            return carry
        lax.fori_loop(0, WB_CHUNK, zb, 0)

        def zacc(q, carry):
            pltpu.sync_copy(
                wb, acc.at[pl.ds(s * NODES_PER_TILE + q * WB_CHUNK, WB_CHUNK)])
            return carry
        lax.fori_loop(0, NODES_PER_TILE // WB_CHUNK, zacc, 0)
        plsc.subcore_barrier()

        base_chunk = s * CHUNKS_PER_TILE

        def stage_desc(sg, arr, buf3):
            return pltpu.make_async_copy(
                edges2.at[arr, pl.ds(base_chunk + sg * SG, SG)],
                buf3.at[lax.rem(sg, 3)], sem_i)

        def stage(sg):
            stage_desc(sg, gd, gidx3).start()
            stage_desc(sg, sd, sidx3).start()

        def wait_stage(sg):
            stage_desc(sg, gd, gidx3).wait()
            stage_desc(sg, sd, sidx3).wait()

        def gather_desc(t, slot):
            sgs = lax.rem(t // SG, 3)
            jof = lax.rem(t, SG)
            return pltpu.make_async_copy(
                tabs.at[gd].at[gidx3.at[sgs, jof]], rows.at[slot],
                sems_g[slot])

        def scatter_desc(t, slot):
            sgs = lax.rem(t // SG, 3)
            jof = lax.rem(t, SG)
            return pltpu.make_async_copy(
                rows.at[slot], acc.at[sidx3.at[sgs, jof]], sems_s[slot])

        pltpu.sync_copy(edges2.at[gd, pl.ds(base_chunk, SG)], gidx3.at[0])
        pltpu.sync_copy(edges2.at[sd, pl.ds(base_chunk, SG)], sidx3.at[0])
        stage(1)
        for t0 in range(LOOKAHEAD):
            gather_desc(t0, t0).start()

        def outer(it, carry):
            for k in range(NSLOT):
                t = it * NSLOT + k
                tg = t + LOOKAHEAD
                gslot = (k + LOOKAHEAD) % NSLOT
                if k == 2:
                    def do_stage(tg=tg):
                        sgg = tg // SG
                        wait_stage(sgg)
                        pl.when(sgg + 1 < SUPERS_PER_TILE)(
                            lambda: stage(sgg + 1))
                    pl.when(jnp.logical_and(lax.rem(tg, SG) == 0,
                                            tg < CHUNKS_PER_TILE))(do_stage)
                pl.when(t >= LOOKAHEAD)(
                    lambda t=t, gs=gslot: scatter_desc(t - LOOKAHEAD,
                                                       gs).wait())
                pl.when(tg < CHUNKS_PER_TILE)(
                    lambda tg=tg, gs=gslot: gather_desc(tg, gs).start())
                gather_desc(t, k).wait()
                scatter_desc(t, k).start(add=True)
            return carry
        lax.fori_loop(0, ITERS_PER_TILE, outer, 0)

        for t_tail in range(CHUNKS_PER_TILE - LOOKAHEAD, CHUNKS_PER_TILE):
            scatter_desc(t_tail, t_tail % NSLOT).wait()
        plsc.subcore_barrier()

        def wq(q, carry):
            nb = s * NODES_PER_TILE + q * WB_CHUNK
            pltpu.sync_copy(acc.at[pl.ds(nb, WB_CHUNK)], wb)
            pltpu.sync_copy(recs.at[pl.ds(sd * N_PAD + nb, WB_CHUNK)], rec_buf)
            if final_mode:
                pltpu.sync_copy(bases.at[sd, pl.ds(nb, WB_CHUNK)], h0b)
                pltpu.sync_copy(tabs.at[sd, pl.ds(nb, WB_CHUNK)], h1b)

            def wn(j, carry2):
                rv = rec_buf[pl.ds(j * LANES, LANES)]
                for k in range(LANES):
                    n = j * LANES + k
                    r = rv[k]
                    v0 = wb[n, pl.ds(0, LANES)] * r
                    v1 = wb[n, pl.ds(LANES, LANES)] * r
                    if final_mode:
                        third = _f32(1.0 / 3.0)
                        v0 = (v0 + h0b[n, pl.ds(0, LANES)]
                              + h1b[n, pl.ds(0, LANES)]) * third
                        v1 = (v1 + h0b[n, pl.ds(LANES, LANES)]
                              + h1b[n, pl.ds(LANES, LANES)]) * third
                    wb[n, pl.ds(0, LANES)] = v0
                    wb[n, pl.ds(LANES, LANES)] = v1
                return carry2
            lax.fori_loop(0, WB_CHUNK // LANES, wn, 0)
            pltpu.sync_copy(wb, outs.at[sd, pl.ds(nb, WB_CHUNK)])
            return carry
        lax.fori_loop(0, NODES_PER_TILE // WB_CHUNK, wq, 0)

    return body


_deg_kernel = pl.kernel(
    _deg_body,
    out_type=jax.ShapeDtypeStruct((2 * N_PAD,), _f32),
    mesh=_mesh,
    compiler_params=pltpu.CompilerParams(use_tc_tiling_on_sc=False),
    scratch_types=[
        pltpu.VMEM((DEG_GROUP, CHUNK), jnp.int32),
        pltpu.VMEM((CHUNK,), jnp.float32),
        pltpu.VMEM((NODES_PER_TILE,), jnp.float32),
        pltpu.VMEM((NODES_PER_TILE,), jnp.float32),
        pltpu.VMEM_SHARED((N_PAD,), jnp.float32),
        pltpu.SemaphoreType.DMA,
    ],
)

_prop_scratch = [
    pltpu.VMEM_SHARED((N_PAD, EMB), jnp.float32),
    pltpu.VMEM((3, SG, CHUNK), jnp.int32),
    pltpu.VMEM((3, SG, CHUNK), jnp.int32),
    pltpu.VMEM((NSLOT, CHUNK, EMB), jnp.float32),
    pltpu.VMEM((WB_CHUNK, EMB), jnp.float32),
    pltpu.VMEM((WB_CHUNK,), jnp.float32),
]
_prop_sems = [pltpu.SemaphoreType.DMA] * (1 + 2 * NSLOT)

_prop_kernel = pl.kernel(
    _make_prop_body(False),
    out_type=jax.ShapeDtypeStruct((2, N_PAD, EMB), _f32),
    mesh=_mesh,
    compiler_params=pltpu.CompilerParams(use_tc_tiling_on_sc=False),
    scratch_types=_prop_scratch + _prop_sems,
)

_final_kernel = pl.kernel(
    _make_prop_body(True),
    out_type=jax.ShapeDtypeStruct((2, N_PAD, EMB), _f32),
    mesh=_mesh,
    compiler_params=pltpu.CompilerParams(use_tc_tiling_on_sc=False),
    scratch_types=_prop_scratch + [
        pltpu.VMEM((WB_CHUNK, EMB), jnp.float32),
        pltpu.VMEM((WB_CHUNK, EMB), jnp.float32),
    ] + _prop_sems,
)


def kernel(edge_index, user_emb, item_emb):
    pad = jnp.full((2, E_PAD - N_EDGES), PAD_NODE, jnp.int32)
    edges2 = jnp.concatenate([edge_index, pad], axis=1)
    edges2 = edges2.reshape(2, N_CHUNKS, CHUNK)
    zrow = jnp.zeros((1, N_PAD - N_USER, EMB), _f32)
    tabs0 = jnp.concatenate(
        [jnp.stack([user_emb, item_emb]),
         jnp.concatenate([zrow, zrow])], axis=1)

    recs = _deg_kernel(edges2)
    tabs1 = _prop_kernel(edges2, tabs0, recs)
    outs = _final_kernel(edges2, tabs1, recs, tabs0)
    return outs[0, :N_USER], outs[1, :N_ITEM]
